# baseline (speedup 1.0000x reference)
import functools

import jax
import jax.numpy as jnp
from jax import lax
from jax.experimental import pallas as pl
from jax.experimental.layout import Layout
from jax.experimental.pallas import tpu as pltpu
from jax.experimental.pallas import tpu_sc as plsc


def _sc_layout(x):
  lay = Layout(major_to_minor=tuple(range(x.ndim)), tiling=((8,),))
  return jax.experimental.layout.with_layout_constraint(x, lay)


_NC = 2
_NS = 16
_NW = _NC * _NS
_CH = 128
_SUB = 8


def _build(B, V, D):
  b_per_w = B // _NW
  spr = D // _SUB
  n_st = b_per_w * spr // _CH
  n_bch = b_per_w // _CH

  mesh = plsc.VectorSubcoreMesh(
      core_axis_name="c", subcore_axis_name="s",
      num_cores=_NC, num_subcores=_NS)

  @functools.partial(
      pl.kernel,
      mesh=mesh,
      compiler_params=pltpu.CompilerParams(use_tc_tiling_on_sc=False),
      out_type=[
          jax.ShapeDtypeStruct((_NW, n_st, _CH, _SUB), jnp.float32),
          jax.ShapeDtypeStruct((_NW, n_st, _CH, _SUB), jnp.float32),
          jax.ShapeDtypeStruct((_NW, n_bch, _CH), jnp.float32),
          jax.ShapeDtypeStruct((_NW, n_bch, _CH), jnp.float32),
      ],
      scratch_types=[
          pltpu.VMEM((n_st, _CH), jnp.int32),
          pltpu.VMEM((n_st, _CH), jnp.int32),
          pltpu.VMEM((n_bch, _CH), jnp.int32),
          pltpu.VMEM((n_bch, _CH), jnp.int32),
          pltpu.VMEM((n_st, _CH, _SUB), jnp.float32),
          pltpu.VMEM((n_st, _CH, _SUB), jnp.float32),
          pltpu.VMEM((n_bch, _CH), jnp.float32),
          pltpu.VMEM((n_bch, _CH), jnp.float32),
          pltpu.SemaphoreType.DMA,
      ],
  )
  def glove_gather(sub_i_hbm, sub_j_hbm, idx_i_hbm, idx_j_hbm,
                   wi_hbm, wj_hbm, bi_hbm, bj_hbm,
                   out_wi, out_wj, out_bi, out_bj,
                   sub_i, sub_j, idx_i, idx_j,
                   rows_i, rows_j, bv_i, bv_j, sem):
    wid = lax.axis_index("s") * _NC + lax.axis_index("c")
    pltpu.sync_copy(sub_i_hbm.at[wid], sub_i)
    pltpu.sync_copy(sub_j_hbm.at[wid], sub_j)
    pltpu.sync_copy(idx_i_hbm.at[wid], idx_i)
    pltpu.sync_copy(idx_j_hbm.at[wid], idx_j)
    handles = []
    for s in range(n_st):
      handles.append(
          pltpu.async_copy(wi_hbm.at[sub_i.at[s]], rows_i.at[s], sem))
      handles.append(
          pltpu.async_copy(wj_hbm.at[sub_j.at[s]], rows_j.at[s], sem))
    for c in range(n_bch):
      handles.append(
          pltpu.async_copy(bi_hbm.at[idx_i.at[c]], bv_i.at[c], sem))
      handles.append(
          pltpu.async_copy(bj_hbm.at[idx_j.at[c]], bv_j.at[c], sem))
    for h in handles:
      h.wait()
    pltpu.sync_copy(rows_i, out_wi.at[wid])
    pltpu.sync_copy(rows_j, out_wj.at[wid])
    pltpu.sync_copy(bv_i, out_bi.at[wid])
    pltpu.sync_copy(bv_j, out_bj.at[wid])

  return glove_gather, n_st, n_bch


@jax.jit
def kernel(new_wi, new_wj, wi, wj, bi, bj):
  B = new_wi.shape[0]
  V, D = wi.shape
  fn, n_st, n_bch = _build(B, V, D)
  sub = jnp.arange(_SUB, dtype=jnp.int32)[None, :]
  sub_i = (new_wi[:, None] * _SUB + sub).reshape(_NW, n_st, _CH)
  sub_j = (new_wj[:, None] * _SUB + sub).reshape(_NW, n_st, _CH)
  out_wi, out_wj, out_bi, out_bj = fn(
      _sc_layout(sub_i), _sc_layout(sub_j),
      _sc_layout(new_wi.reshape(_NW, n_bch, _CH)),
      _sc_layout(new_wj.reshape(_NW, n_bch, _CH)),
      _sc_layout(wi).reshape(V * _SUB, D // _SUB),
      _sc_layout(wj).reshape(V * _SUB, D // _SUB),
      _sc_layout(bi.reshape(V)), _sc_layout(bj.reshape(V)))
  return (out_wi.reshape(B, D), out_wj.reshape(B, D),
          out_bi.reshape(B), out_bj.reshape(B))


# speedup vs baseline: 3.9332x; 3.9332x over previous
import functools

import jax
import jax.numpy as jnp
from jax import lax
from jax.experimental import pallas as pl
from jax.experimental.layout import Layout
from jax.experimental.pallas import tpu as pltpu
from jax.experimental.pallas import tpu_sc as plsc


def _tiled_layout(x):
  lay = Layout(major_to_minor=tuple(range(x.ndim)), tiling=((8, 128),))
  return jax.experimental.layout.with_layout_constraint(x, lay)


_NC = 2
_NS = 16
_NW = _NC * _NS
_CH = 128
_NPH = 2


def _build(B, V, D):
  b_per_w = B // _NW
  n_st = b_per_w * D // _CH
  st_ph = n_st // _NPH
  n_bch = b_per_w // _CH

  mesh = plsc.VectorSubcoreMesh(
      core_axis_name="c", subcore_axis_name="s",
      num_cores=_NC, num_subcores=_NS)

  @functools.partial(
      pl.kernel,
      mesh=mesh,
      compiler_params=pltpu.CompilerParams(use_tc_tiling_on_sc=True),
      out_type=[
          jax.ShapeDtypeStruct((_NW, _NPH, st_ph, _CH), jnp.float32),
          jax.ShapeDtypeStruct((_NW, _NPH, st_ph, _CH), jnp.float32),
          jax.ShapeDtypeStruct((_NW, 8, _CH), jnp.float32),
          jax.ShapeDtypeStruct((_NW, 8, _CH), jnp.float32),
      ],
      scratch_types=[
          pltpu.VMEM((st_ph, _CH), jnp.int32),
          pltpu.VMEM((st_ph, _CH), jnp.int32),
          pltpu.VMEM((8, _CH), jnp.int32),
          pltpu.VMEM((8, _CH), jnp.int32),
          pltpu.VMEM((st_ph, _CH), jnp.float32),
          pltpu.VMEM((st_ph, _CH), jnp.float32),
          pltpu.VMEM((8, _CH), jnp.float32),
          pltpu.VMEM((8, _CH), jnp.float32),
          pltpu.SemaphoreType.DMA,
          pltpu.SemaphoreType.DMA,
      ],
  )
  def glove_gather(off_i_hbm, off_j_hbm, idx_i_hbm, idx_j_hbm,
                   wit_hbm, wjt_hbm, bi_hbm, bj_hbm,
                   out_wi, out_wj, out_bi, out_bj,
                   off_i, off_j, idx_i, idx_j,
                   dst_i, dst_j, bv_i, bv_j, sem, bsem):
    wid = lax.axis_index("s") * _NC + lax.axis_index("c")
    pltpu.sync_copy(idx_i_hbm.at[wid], idx_i)
    pltpu.sync_copy(idx_j_hbm.at[wid], idx_j)
    wi_base = wit_hbm.at[0]
    wj_base = wjt_hbm.at[0]
    bias_handles = []
    for c in range(n_bch):
      bias_handles.append(
          pltpu.async_copy(bi_hbm.at[idx_i.at[c]], bv_i.at[c], bsem))
      bias_handles.append(
          pltpu.async_copy(bj_hbm.at[idx_j.at[c]], bv_j.at[c], bsem))
    for ph in range(_NPH):
      pltpu.sync_copy(off_i_hbm.at[wid].at[ph], off_i)
      pltpu.sync_copy(off_j_hbm.at[wid].at[ph], off_j)

      def fire(s, _):
        pltpu.async_copy(wi_base.at[off_i.at[s]], dst_i.at[s], sem)
        pltpu.async_copy(wj_base.at[off_j.at[s]], dst_j.at[s], sem)
        return ()

      lax.fori_loop(0, st_ph, fire, (), unroll=False)
      pltpu.make_async_copy(out_wi.at[wid].at[ph], dst_i, sem).wait()
      pltpu.make_async_copy(out_wj.at[wid].at[ph], dst_j, sem).wait()
      pltpu.sync_copy(dst_i, out_wi.at[wid].at[ph])
      pltpu.sync_copy(dst_j, out_wj.at[wid].at[ph])
    for h in bias_handles:
      h.wait()
    pltpu.sync_copy(bv_i, out_bi.at[wid])
    pltpu.sync_copy(bv_j, out_bj.at[wid])

  return glove_gather, n_st, n_bch


def _phys_offsets(idx, V, D):
  tiles_r = (V + 127) // 128
  c = jnp.arange(D, dtype=jnp.int32)[None, :]
  r = idx[:, None]
  return ((c // 8) * (tiles_r * 1024) + (r // 128) * 1024
          + (c % 8) * 128 + (r % 128))


@jax.jit
def kernel(new_wi, new_wj, wi, wj, bi, bj):
  B = new_wi.shape[0]
  V, D = wi.shape
  fn, n_st, n_bch = _build(B, V, D)
  st_ph = n_st // _NPH
  off_i = _phys_offsets(new_wi, V, D).reshape(_NW, _NPH, st_ph, _CH)
  off_j = _phys_offsets(new_wj, V, D).reshape(_NW, _NPH, st_ph, _CH)
  idx_i = jnp.pad(new_wi.reshape(_NW, n_bch, _CH),
                  ((0, 0), (0, 8 - n_bch), (0, 0)))
  idx_j = jnp.pad(new_wj.reshape(_NW, n_bch, _CH),
                  ((0, 0), (0, 8 - n_bch), (0, 0)))
  out_wi, out_wj, out_bi, out_bj = fn(
      off_i, off_j, idx_i, idx_j,
      _tiled_layout(wi.T), _tiled_layout(wj.T),
      bi.reshape(V), bj.reshape(V))
  out_bi = out_bi[:, :n_bch, :].reshape(B)
  out_bj = out_bj[:, :n_bch, :].reshape(B)
  return (out_wi.reshape(B, D), out_wj.reshape(B, D), out_bi, out_bj)


# speedup vs baseline: 5.4717x; 1.3911x over previous
import functools

import jax
import jax.numpy as jnp
from jax import lax
from jax.experimental import pallas as pl
from jax.experimental.layout import Layout
from jax.experimental.pallas import tpu as pltpu
from jax.experimental.pallas import tpu_sc as plsc


def _tiled_layout(x):
  lay = Layout(major_to_minor=tuple(range(x.ndim)), tiling=((8, 128),))
  return jax.experimental.layout.with_layout_constraint(x, lay)


_NC = 2
_NS = 16
_NW = _NC * _NS
_CH = 128
_NPH = 2


def _build(B, V, D):
  b_per_w = B // _NW
  n_ch = b_per_w // _CH
  d_ph = D // _NPH

  mesh = plsc.VectorSubcoreMesh(
      core_axis_name="c", subcore_axis_name="s",
      num_cores=_NC, num_subcores=_NS)

  @functools.partial(
      pl.kernel,
      mesh=mesh,
      compiler_params=pltpu.CompilerParams(use_tc_tiling_on_sc=True),
      out_type=[
          jax.ShapeDtypeStruct((D, B // _CH, _CH), jnp.float32),
          jax.ShapeDtypeStruct((D, B // _CH, _CH), jnp.float32),
          jax.ShapeDtypeStruct((_NW, 8, _CH), jnp.float32),
          jax.ShapeDtypeStruct((_NW, 8, _CH), jnp.float32),
      ],
      scratch_types=[
          pltpu.VMEM((d_ph, n_ch, _CH), jnp.int32),
          pltpu.VMEM((d_ph, n_ch, _CH), jnp.int32),
          pltpu.VMEM((8, _CH), jnp.int32),
          pltpu.VMEM((8, _CH), jnp.int32),
          pltpu.VMEM((d_ph, n_ch, _CH), jnp.float32),
          pltpu.VMEM((d_ph, n_ch, _CH), jnp.float32),
          pltpu.VMEM((8, _CH), jnp.float32),
          pltpu.VMEM((8, _CH), jnp.float32),
          pltpu.SemaphoreType.DMA,
          pltpu.SemaphoreType.DMA,
      ],
  )
  def glove_gather(off_i_hbm, off_j_hbm, idx_i_hbm, idx_j_hbm,
                   wit_hbm, wjt_hbm, bi_hbm, bj_hbm,
                   out_wi, out_wj, out_bi, out_bj,
                   off_i, off_j, idx_i, idx_j,
                   dst_i, dst_j, bv_i, bv_j, sem, bsem):
    wid = lax.axis_index("s") * _NC + lax.axis_index("c")
    pltpu.sync_copy(idx_i_hbm.at[wid], idx_i)
    pltpu.sync_copy(idx_j_hbm.at[wid], idx_j)
    wi_base = wit_hbm.at[0]
    wj_base = wjt_hbm.at[0]
    bias_handles = []
    for c in range(n_ch):
      bias_handles.append(
          pltpu.async_copy(bi_hbm.at[idx_i.at[c]], bv_i.at[c], bsem))
      bias_handles.append(
          pltpu.async_copy(bj_hbm.at[idx_j.at[c]], bv_j.at[c], bsem))
    bsel = pl.ds(wid * n_ch, n_ch)
    for ph in range(_NPH):
      fsel = pl.ds(ph * d_ph, d_ph)
      pltpu.sync_copy(off_i_hbm.at[fsel, bsel], off_i)
      pltpu.sync_copy(off_j_hbm.at[fsel, bsel], off_j)

      def fire(s, _):
        c = s // n_ch
        ch = s % n_ch
        pltpu.async_copy(wi_base.at[off_i.at[c].at[ch]],
                         dst_i.at[c].at[ch], sem)
        pltpu.async_copy(wj_base.at[off_j.at[c].at[ch]],
                         dst_j.at[c].at[ch], sem)
        return ()

      lax.fori_loop(0, d_ph * n_ch, fire, (), unroll=False)
      pltpu.make_async_copy(out_wi.at[fsel, bsel], dst_i, sem).wait()
      pltpu.make_async_copy(out_wj.at[fsel, bsel], dst_j, sem).wait()
      pltpu.sync_copy(dst_i, out_wi.at[fsel, bsel])
      pltpu.sync_copy(dst_j, out_wj.at[fsel, bsel])
    for h in bias_handles:
      h.wait()
    pltpu.sync_copy(bv_i, out_bi.at[wid])
    pltpu.sync_copy(bv_j, out_bj.at[wid])

  return glove_gather, n_ch


@jax.jit
def kernel(new_wi, new_wj, wi, wj, bi, bj):
  B = new_wi.shape[0]
  V, D = wi.shape
  fn, n_ch = _build(B, V, D)
  tiles_r = (V + 127) // 128
  c = jnp.arange(D, dtype=jnp.int32)
  col_term = (c // 8) * (tiles_r * 1024) + (c % 8) * 128
  row_term_i = (new_wi // 128) * 1024 + (new_wi % 128)
  row_term_j = (new_wj // 128) * 1024 + (new_wj % 128)
  off_i = (col_term[:, None] + row_term_i[None, :]).reshape(D, B // _CH, _CH)
  off_j = (col_term[:, None] + row_term_j[None, :]).reshape(D, B // _CH, _CH)
  idx_i = jnp.pad(new_wi.reshape(_NW, n_ch, _CH),
                  ((0, 0), (0, 8 - n_ch), (0, 0)))
  idx_j = jnp.pad(new_wj.reshape(_NW, n_ch, _CH),
                  ((0, 0), (0, 8 - n_ch), (0, 0)))
  out_wi, out_wj, out_bi, out_bj = fn(
      off_i, off_j, idx_i, idx_j,
      _tiled_layout(wi.T), _tiled_layout(wj.T),
      bi.reshape(V), bj.reshape(V))
  out_bi = out_bi[:, :n_ch, :].reshape(B)
  out_bj = out_bj[:, :n_ch, :].reshape(B)
  return (out_wi.reshape(D, B).T, out_wj.reshape(D, B).T, out_bi, out_bj)


# speedup vs baseline: 8.9242x; 1.6310x over previous
import functools

import jax
import jax.numpy as jnp
from jax import lax
from jax.experimental import pallas as pl
from jax.experimental.layout import Layout
from jax.experimental.pallas import tpu as pltpu
from jax.experimental.pallas import tpu_sc as plsc


def _tiled_layout(x):
  lay = Layout(major_to_minor=tuple(range(x.ndim)), tiling=((8, 128),))
  return jax.experimental.layout.with_layout_constraint(x, lay)


def _bias_layout(x):
  lay = Layout(major_to_minor=tuple(range(x.ndim)), tiling=((1, 128),))
  return jax.experimental.layout.with_layout_constraint(x, lay)


_NC = 2
_NS = 16
_NW = _NC * _NS
_CH = 128
_NPH = 2


def _build(B, V, D):
  b_per_w = B // _NW
  n_ch = b_per_w // _CH
  d_ph = D // _NPH

  mesh = plsc.VectorSubcoreMesh(
      core_axis_name="c", subcore_axis_name="s",
      num_cores=_NC, num_subcores=_NS)

  @functools.partial(
      pl.kernel,
      mesh=mesh,
      compiler_params=pltpu.CompilerParams(use_tc_tiling_on_sc=True),
      out_type=[
          jax.ShapeDtypeStruct((D, B // _CH, _CH), jnp.float32),
          jax.ShapeDtypeStruct((D, B // _CH, _CH), jnp.float32),
          jax.ShapeDtypeStruct((_NW, 4, _CH), jnp.float32),
          jax.ShapeDtypeStruct((_NW, 4, _CH), jnp.float32),
      ],
      scratch_types=[
          pltpu.VMEM((d_ph, n_ch, _CH), jnp.int32),
          pltpu.VMEM((d_ph, n_ch, _CH), jnp.int32),
          pltpu.VMEM((4, _CH), jnp.int32),
          pltpu.VMEM((4, _CH), jnp.int32),
          pltpu.VMEM((d_ph, n_ch, _CH), jnp.float32),
          pltpu.VMEM((d_ph, n_ch, _CH), jnp.float32),
          pltpu.VMEM((4, _CH), jnp.float32),
          pltpu.VMEM((4, _CH), jnp.float32),
          pltpu.SemaphoreType.DMA,
          pltpu.SemaphoreType.DMA,
      ],
  )
  def glove_gather(off_i_hbm, off_j_hbm, idx_i_hbm, idx_j_hbm,
                   wit_hbm, wjt_hbm, bi_hbm, bj_hbm,
                   out_wi, out_wj, out_bi, out_bj,
                   off_i, off_j, idx_i, idx_j,
                   dst_i, dst_j, bv_i, bv_j, sem, bsem):
    wid = lax.axis_index("s") * _NC + lax.axis_index("c")
    pltpu.sync_copy(idx_i_hbm.at[wid], idx_i)
    pltpu.sync_copy(idx_j_hbm.at[wid], idx_j)
    wi_base = wit_hbm.at[0]
    wj_base = wjt_hbm.at[0]
    bias_handles = []
    for c in range(n_ch):
      bias_handles.append(
          pltpu.async_copy(bi_hbm.at[0].at[idx_i.at[c]], bv_i.at[c], bsem))
      bias_handles.append(
          pltpu.async_copy(bj_hbm.at[0].at[idx_j.at[c]], bv_j.at[c], bsem))
    bsel = pl.ds(wid * n_ch, n_ch)
    for ph in range(_NPH):
      fsel = pl.ds(ph * d_ph, d_ph)
      pltpu.sync_copy(off_i_hbm.at[fsel, bsel], off_i)
      pltpu.sync_copy(off_j_hbm.at[fsel, bsel], off_j)

      def fire(s, _):
        c = s // n_ch
        ch = s % n_ch
        pltpu.async_copy(wi_base.at[off_i.at[c].at[ch]],
                         dst_i.at[c].at[ch], sem)
        pltpu.async_copy(wj_base.at[off_j.at[c].at[ch]],
                         dst_j.at[c].at[ch], sem)
        return ()

      lax.fori_loop(0, d_ph * n_ch, fire, (), unroll=False)
      pltpu.make_async_copy(out_wi.at[fsel, bsel], dst_i, sem).wait()
      pltpu.make_async_copy(out_wj.at[fsel, bsel], dst_j, sem).wait()
      pltpu.sync_copy(dst_i, out_wi.at[fsel, bsel])
      pltpu.sync_copy(dst_j, out_wj.at[fsel, bsel])
    for h in bias_handles:
      h.wait()
    pltpu.sync_copy(bv_i, out_bi.at[wid])
    pltpu.sync_copy(bv_j, out_bj.at[wid])

  return glove_gather, n_ch


@jax.jit
def kernel(new_wi, new_wj, wi, wj, bi, bj):
  B = new_wi.shape[0]
  V, D = wi.shape
  fn, n_ch = _build(B, V, D)
  tiles_r = (V + 127) // 128
  c = jnp.arange(D, dtype=jnp.int32)
  col_term = (c // 8) * (tiles_r * 1024) + (c % 8) * 128
  row_term_i = (new_wi // 128) * 1024 + (new_wi % 128)
  row_term_j = (new_wj // 128) * 1024 + (new_wj % 128)
  off_i = (col_term[:, None] + row_term_i[None, :]).reshape(D, B // _CH, _CH)
  off_j = (col_term[:, None] + row_term_j[None, :]).reshape(D, B // _CH, _CH)
  idx_i = new_wi.reshape(_NW, n_ch, _CH)
  idx_j = new_wj.reshape(_NW, n_ch, _CH)
  out_wi, out_wj, out_bi, out_bj = fn(
      off_i, off_j, idx_i, idx_j,
      _tiled_layout(wi.T), _tiled_layout(wj.T),
      _bias_layout(bi.T), _bias_layout(bj.T))
  out_bi = out_bi.reshape(B)
  out_bj = out_bj.reshape(B)
  return (out_wi.reshape(D, B).T, out_wj.reshape(D, B).T, out_bi, out_bj)
